# baseline (speedup 1.0000x reference)
import functools

import jax
import jax.numpy as jnp
from jax.experimental import pallas as pl
from jax.experimental.pallas import tpu as pltpu

U = 25000
I = 25000
D = 64
B = 16384
N_LAYERS = 3
N_SOC_LAYERS = 2
ALPHA = 0.6
BETA = 0.4


def _gbgcn_layer(x, src, dst, ew, Ws, bs, Wn, bn, bias):
    N = x.shape[0]
    loop = jnp.arange(N)
    s = jnp.concatenate([src, loop])
    d = jnp.concatenate([dst, loop])
    if ew is None:
        msg = x[s]
    else:
        w = jnp.concatenate([ew, jnp.ones((N,), ew.dtype)])
        msg = x[s] * w[:, None]
    summed = jax.ops.segment_sum(msg, d, num_segments=N)
    cnt = jax.ops.segment_sum(jnp.ones((s.shape[0],), x.dtype), d, num_segments=N)
    agg = summed / jnp.maximum(cnt, 1.0)[:, None]
    out = x @ Ws.T + bs + agg @ Wn.T + bn + bias
    return jax.nn.leaky_relu(out, 0.2)


def _mlp_body(feat_ref, g_ref, w1_ref, b1_ref, w2_ref, b2_ref, w3_ref, b3_ref,
              gw1_ref, gb1_ref, gw2_ref, gb2_ref, out_ref):
    feat = feat_ref[...]
    h = jnp.maximum(feat @ w1_ref[...] + b1_ref[...], 0.0)
    h = jnp.maximum(h @ w2_ref[...] + b2_ref[...], 0.0)
    pred = jax.nn.sigmoid(h @ w3_ref[...] + b3_ref[...])
    g = g_ref[...]
    gh = jnp.maximum(g @ gw1_ref[...] + gb1_ref[...], 0.0)
    group = jax.nn.sigmoid(gh @ gw2_ref[...] + gb2_ref[...])
    out_ref[...] = jnp.concatenate([pred, group], axis=1)


def _mlp_heads(feat, g, p):
    blk = 2048
    grid = (B // blk,)
    full = lambda shape: pl.BlockSpec(shape, lambda i: (0, 0))
    return pl.pallas_call(
        _mlp_body,
        grid=grid,
        in_specs=[
            pl.BlockSpec((blk, 4 * D), lambda i: (i, 0)),
            pl.BlockSpec((blk, 2 * D), lambda i: (i, 0)),
            full((4 * D, 2 * D)), full((1, 2 * D)),
            full((2 * D, D)), full((1, D)),
            full((D, 1)), full((1, 1)),
            full((2 * D, D)), full((1, D)),
            full((D, 1)), full((1, 1)),
        ],
        out_specs=pl.BlockSpec((blk, 2), lambda i: (i, 0)),
        out_shape=jax.ShapeDtypeStruct((B, 2), jnp.float32),
    )(feat, g,
      p["p_W1"].T, p["p_b1"][None, :],
      p["p_W2"].T, p["p_b2"][None, :],
      p["p_W3"].T, p["p_b3"][None, :],
      p["g_W1"].T, p["g_b1"][None, :],
      p["g_W2"].T, p["g_b2"][None, :])


def kernel(params, social_edge_weights, user_ids, item_ids,
           initiator_edge_index, participant_edge_index, social_edge_index):
    p = params
    all_emb = jnp.concatenate([p["user_table"], p["item_table"]], axis=0)
    x = all_emb
    for l in range(N_LAYERS):
        x = _gbgcn_layer(x, initiator_edge_index[0], initiator_edge_index[1], None,
                         p["init_Ws"][l], p["init_bs"][l], p["init_Wn"][l], p["init_bn"][l], p["init_bias"][l])
    init_emb = x
    x = all_emb
    for l in range(N_LAYERS):
        x = _gbgcn_layer(x, participant_edge_index[0], participant_edge_index[1], None,
                         p["part_Ws"][l], p["part_bs"][l], p["part_Wn"][l], p["part_bn"][l], p["part_bias"][l])
    part_emb = x
    init_u, init_i = init_emb[:U], init_emb[U:]
    part_u, part_i = part_emb[:U], part_emb[U:]
    ai = init_u @ p["cv_Wai"].T + p["cv_bai"]
    ap = part_u @ p["cv_Wap"].T + p["cv_bap"]
    aw = jax.nn.sigmoid(jnp.concatenate([ai, ap], axis=1) @ p["cv_Wc"].T + p["cv_bc"])
    upd_init_u = init_u + aw * (part_u @ p["cv_Wx"].T + p["cv_bx"])
    upd_part_u = part_u + (1.0 - aw) * (init_u @ p["cv_Wx"].T + p["cv_bx"])
    s = p["user_table"]
    for l in range(N_SOC_LAYERS):
        s = _gbgcn_layer(s, social_edge_index[0], social_edge_index[1], social_edge_weights,
                         p["soc_Ws"][l], p["soc_bs"][l], p["soc_Wn"][l], p["soc_bn"][l], p["soc_bias"][l])
    soc_inf = s @ p["soc_agg_W"].T + p["soc_agg_b"]
    item_final = ALPHA * init_i + BETA * part_i
    fu = upd_init_u[user_ids]
    feat = jnp.concatenate([fu, upd_part_u[user_ids], soc_inf[user_ids], item_final[item_ids]], axis=1)
    g = jnp.concatenate([fu, item_final[item_ids]], axis=1)
    return _mlp_heads(feat, g, p)


# speedup vs baseline: 5.0011x; 5.0011x over previous
import functools

import jax
import jax.numpy as jnp
from jax import lax
from jax.experimental import pallas as pl
from jax.experimental.pallas import tpu as pltpu
from jax.experimental.pallas import tpu_sc as plsc

U = 25000
I = 25000
D = 64
B = 16384
N_LAYERS = 3
N_SOC_LAYERS = 2
ALPHA = 0.6
BETA = 0.4

N_ALL = U + I
NPAD = 50176
UPAD = 25088
EPV = 802816
EPS = 409600

_f32 = jnp.float32
_i32 = jnp.int32


@functools.cache
def _segsum_kernel(npad, ep, with_counts, with_weights):
    nsub = 16
    stripe = npad // nsub
    wchunk = stripe // 8
    rows_per_sub = ep // nsub // 128
    kj = 2
    nblk = rows_per_sub // kj

    mesh = plsc.VectorSubcoreMesh(core_axis_name="c", subcore_axis_name="s")

    out_type = [jax.ShapeDtypeStruct((npad, 32), _f32),
                jax.ShapeDtypeStruct((npad, 32), _f32)]
    if with_counts:
        out_type.append(jax.ShapeDtypeStruct((npad,), _f32))

    scratch = [
        pltpu.VMEM((kj, 128), _i32),
        pltpu.VMEM((kj, 128), _i32),
        pltpu.VMEM((kj * 128, 32), _f32),
        pltpu.VMEM((wchunk, 32), _f32),
        pltpu.VMEM_SHARED((npad, 32), _f32),
        pltpu.SemaphoreType.DMA,
    ]
    if with_counts:
        scratch += [
            pltpu.VMEM((kj, 128), _f32),
            pltpu.VMEM_SHARED((npad,), _f32),
            pltpu.VMEM((stripe,), _f32),
        ]
    if with_weights:
        scratch.append(pltpu.VMEM((kj, 128), _f32))

    def body(*refs):
        it = iter(refs)
        xa = next(it); xb = next(it); src2 = next(it); dst2 = next(it)
        w2 = next(it) if with_weights else None
        ga = next(it); gb = next(it)
        cnt = next(it) if with_counts else None
        src_v = next(it); dst_v = next(it); rows_v = next(it); zb = next(it)
        acc = next(it); sem = next(it)
        if with_counts:
            ones_v = next(it); cacc = next(it); ctmp = next(it)
        if with_weights:
            wts = next(it)

        cid = lax.axis_index("c")
        sid = lax.axis_index("s")

        zeros16 = jnp.zeros((16,), _f32)

        def zrow(r, carry):
            zb[r, pl.ds(0, 16)] = zeros16
            zb[r, pl.ds(16, 16)] = zeros16
            return carry
        lax.fori_loop(0, wchunk, zrow, 0)
        for t in range(8):
            pltpu.sync_copy(zb, acc.at[pl.ds(sid * stripe + t * wchunk, wchunk)])

        if with_counts:
            def czrow(r, carry):
                ctmp[pl.ds(r * 16, 16)] = zeros16
                return carry
            lax.fori_loop(0, stripe // 16, czrow, 0)
            pltpu.sync_copy(ctmp, cacc.at[pl.ds(sid * stripe, stripe)])
            ones16 = jnp.ones((16,), _f32)
            for r in range(kj):
                for q in range(8):
                    ones_v[r, pl.ds(q * 16, 16)] = ones16

        plsc.subcore_barrier()

        def blk_body(b, carry):
            rbase = sid * rows_per_sub + b * kj
            pltpu.sync_copy(src2.at[pl.ds(rbase, kj)], src_v)
            pltpu.sync_copy(dst2.at[pl.ds(rbase, kj)], dst_v)
            if with_weights:
                pltpu.sync_copy(w2.at[pl.ds(rbase, kj)], wts)

            @pl.when(cid == 0)
            def _():
                for j in range(kj):
                    pltpu.async_copy(xa.at[src_v.at[j]],
                                     rows_v.at[pl.ds(j * 128, 128)], sem)

            @pl.when(cid == 1)
            def _():
                for j in range(kj):
                    pltpu.async_copy(xb.at[src_v.at[j]],
                                     rows_v.at[pl.ds(j * 128, 128)], sem)

            pltpu.make_async_copy(xa.at[pl.ds(0, kj * 128)], rows_v, sem).wait()

            if with_weights:
                lanes = lax.iota(_i32, 16)

                def scale(e, carry):
                    jrow = e // 8
                    off = (e % 8) * 16
                    w16 = wts[jrow, pl.ds(off, 16)]
                    ridx = e * 16 + lanes
                    for f in range(32):
                        fidx = jnp.full((16,), f, _i32)
                        col = plsc.load_gather(rows_v, [ridx, fidx])
                        plsc.store_scatter(rows_v, [ridx, fidx], col * w16)
                    return carry
                lax.fori_loop(0, kj * 8, scale, 0)

            for j in range(kj):
                pltpu.sync_copy(rows_v.at[pl.ds(j * 128, 128)],
                                acc.at[dst_v.at[j]], add=True)
            if with_counts:
                for j in range(kj):
                    pltpu.sync_copy(ones_v.at[j], cacc.at[dst_v.at[j]], add=True)
            return carry

        lax.fori_loop(0, nblk, blk_body, 0)

        plsc.subcore_barrier()

        for t in range(8):
            off = sid * stripe + t * wchunk
            pltpu.sync_copy(acc.at[pl.ds(off, wchunk)], zb)

            @pl.when(cid == 0)
            def _():
                pltpu.sync_copy(zb, ga.at[pl.ds(off, wchunk)])

            @pl.when(cid == 1)
            def _():
                pltpu.sync_copy(zb, gb.at[pl.ds(off, wchunk)])

        if with_counts:
            pltpu.sync_copy(cacc.at[pl.ds(sid * stripe, stripe)], ctmp)

            @pl.when(cid == 0)
            def _():
                pltpu.sync_copy(ctmp, cnt.at[pl.ds(sid * stripe, stripe)])

    return pl.kernel(body, out_type=out_type, mesh=mesh,
                     scratch_types=scratch,
                     compiler_params=pltpu.CompilerParams(
                         use_tc_tiling_on_sc=False,
                         needs_layout_passes=False))


@functools.cache
def _batch_gather_kernel():
    mesh = plsc.VectorSubcoreMesh(core_axis_name="c", subcore_axis_name="s")
    out_type = [jax.ShapeDtypeStruct((B, 32), _f32) for _ in range(10)]
    scratch = [
        pltpu.VMEM((8, 128), _i32),
        pltpu.VMEM((8, 128), _i32),
        pltpu.VMEM((1024, 32), _f32),
        pltpu.SemaphoreType.DMA,
    ]

    def body(ui2, it2, uia, uib, upa, upb, sia, sib, ia, ib, pa, pb,
             fua, fub, pua, pub, sua, sub2, iia, iib, pia, pib,
             uid_v, iid_v, rows_v, sem):
        cid = lax.axis_index("c")
        sid = lax.axis_index("s")
        rbase = sid * 8
        pltpu.sync_copy(ui2.at[pl.ds(rbase, 8)], uid_v)
        pltpu.sync_copy(it2.at[pl.ds(rbase, 8)], iid_v)
        for r in range(8):
            for q in range(8):
                iid_v[r, pl.ds(q * 16, 16)] = iid_v[r, pl.ds(q * 16, 16)] + U

        def one(ta, tb, oa, ob, idx_v):
            @pl.when(cid == 0)
            def _():
                for j in range(8):
                    pltpu.async_copy(ta.at[idx_v.at[j]],
                                     rows_v.at[pl.ds(j * 128, 128)], sem)

            @pl.when(cid == 1)
            def _():
                for j in range(8):
                    pltpu.async_copy(tb.at[idx_v.at[j]],
                                     rows_v.at[pl.ds(j * 128, 128)], sem)

            pltpu.make_async_copy(ta.at[pl.ds(0, 1024)], rows_v, sem).wait()

            @pl.when(cid == 0)
            def _():
                pltpu.sync_copy(rows_v, oa.at[pl.ds(sid * 1024, 1024)])

            @pl.when(cid == 1)
            def _():
                pltpu.sync_copy(rows_v, ob.at[pl.ds(sid * 1024, 1024)])

        one(uia, uib, fua, fub, uid_v)
        one(upa, upb, pua, pub, uid_v)
        one(sia, sib, sua, sub2, uid_v)
        one(ia, ib, iia, iib, iid_v)
        one(pa, pb, pia, pib, iid_v)

    return pl.kernel(body, out_type=out_type, mesh=mesh,
                     scratch_types=scratch,
                     compiler_params=pltpu.CompilerParams(
                         use_tc_tiling_on_sc=False))


def _leaky(x):
    return jnp.where(x >= 0, x, 0.2 * x)


def _combine_body(xa, xb, ga, gb, cnt, wsT, wnT, bias, oa, ob):
    x = jnp.concatenate([xa[...], xb[...]], axis=1)
    gs = jnp.concatenate([ga[...], gb[...]], axis=1) + x
    agg = gs / (cnt[...] + 1.0)
    y = _leaky(x @ wsT[...] + agg @ wnT[...] + bias[...])
    oa[...] = y[:, :32]
    ob[...] = y[:, 32:]


def _combine_agg_body(xa, xb, ga, gb, cnt, wsT, wnT, bias, waggT, bagg, oa, ob):
    x = jnp.concatenate([xa[...], xb[...]], axis=1)
    gs = jnp.concatenate([ga[...], gb[...]], axis=1) + x
    agg = gs / (cnt[...] + 1.0)
    y = _leaky(x @ wsT[...] + agg @ wnT[...] + bias[...])
    s = y @ waggT[...] + bagg[...]
    oa[...] = s[:, :32]
    ob[...] = s[:, 32:]


def _tc_combine(xa, xb, ga, gb, cnt1, wsT, wnT, bias, npad, agg_w=None):
    blk = 3136
    grid = (npad // blk,)
    half = lambda: pl.BlockSpec((blk, 32), lambda i: (i, 0))
    full = lambda shape: pl.BlockSpec(shape, lambda i: (0, 0))
    in_specs = [half(), half(), half(), half(),
                pl.BlockSpec((blk, 1), lambda i: (i, 0)),
                full((D, D)), full((D, D)), full((1, D))]
    args = [xa, xb, ga, gb, cnt1, wsT, wnT, bias]
    body = _combine_body
    if agg_w is not None:
        in_specs += [full((D, D)), full((1, D))]
        args += [agg_w[0], agg_w[1]]
        body = _combine_agg_body
    return pl.pallas_call(
        body,
        grid=grid,
        in_specs=in_specs,
        out_specs=[pl.BlockSpec((blk, 32), lambda i: (i, 0))] * 2,
        out_shape=[jax.ShapeDtypeStruct((npad, 32), _f32)] * 2,
    )(*args)


def _cross_body(ia, ib, pa, pb, waiT, bai, wapT, bap, wcT, bc, wxT, bx,
                oia, oib, opa, opb):
    iu = jnp.concatenate([ia[...], ib[...]], axis=1)
    pu = jnp.concatenate([pa[...], pb[...]], axis=1)
    att_i = iu @ waiT[...] + bai[...]
    att_p = pu @ wapT[...] + bap[...]
    z = jnp.concatenate([att_i, att_p], axis=1) @ wcT[...] + bc[...]
    aw = jax.nn.sigmoid(z)
    px = pu @ wxT[...] + bx[...]
    ix = iu @ wxT[...] + bx[...]
    ui = iu + aw * px
    up = pu + (1.0 - aw) * ix
    oia[...] = ui[:, :32]
    oib[...] = ui[:, 32:]
    opa[...] = up[:, :32]
    opb[...] = up[:, 32:]


def _tc_cross(ia, ib, pa, pb, p):
    blk = 3136
    grid = (UPAD // blk,)
    half = lambda: pl.BlockSpec((blk, 32), lambda i: (i, 0))
    full = lambda shape: pl.BlockSpec(shape, lambda i: (0, 0))
    return pl.pallas_call(
        _cross_body,
        grid=grid,
        in_specs=[half(), half(), half(), half(),
                  full((D, D)), full((1, D)),
                  full((D, D)), full((1, D)),
                  full((2 * D, 1)), full((1, 1)),
                  full((D, D)), full((1, D))],
        out_specs=[pl.BlockSpec((blk, 32), lambda i: (i, 0))] * 4,
        out_shape=[jax.ShapeDtypeStruct((UPAD, 32), _f32)] * 4,
    )(ia, ib, pa, pb,
      p["cv_Wai"].T, p["cv_bai"][None, :],
      p["cv_Wap"].T, p["cv_bap"][None, :],
      p["cv_Wc"].T, p["cv_bc"][None, :],
      p["cv_Wx"].T, p["cv_bx"][None, :])


def _head_body(fua, fub, pua, pub, sua, sub2, iia, iib, pia, pib,
               w1, b1, w2, b2, w3, b3, gw1, gb1, gw2, gb2, out):
    fu = jnp.concatenate([fua[...], fub[...]], axis=1)
    pu = jnp.concatenate([pua[...], pub[...]], axis=1)
    su = jnp.concatenate([sua[...], sub2[...]], axis=1)
    itf = ALPHA * jnp.concatenate([iia[...], iib[...]], axis=1) + \
        BETA * jnp.concatenate([pia[...], pib[...]], axis=1)
    feat = jnp.concatenate([fu, pu, su, itf], axis=1)
    h = jnp.maximum(feat @ w1[...] + b1[...], 0.0)
    h = jnp.maximum(h @ w2[...] + b2[...], 0.0)
    pred = jax.nn.sigmoid(h @ w3[...] + b3[...])
    g = jnp.concatenate([fu, itf], axis=1)
    gh = jnp.maximum(g @ gw1[...] + gb1[...], 0.0)
    group = jax.nn.sigmoid(gh @ gw2[...] + gb2[...])
    out[...] = jnp.concatenate([pred, group], axis=1)


def _tc_head(gathered, p):
    blk = 2048
    grid = (B // blk,)
    half = lambda: pl.BlockSpec((blk, 32), lambda i: (i, 0))
    full = lambda shape: pl.BlockSpec(shape, lambda i: (0, 0))
    return pl.pallas_call(
        _head_body,
        grid=grid,
        in_specs=[half() for _ in range(10)] + [
            full((4 * D, 2 * D)), full((1, 2 * D)),
            full((2 * D, D)), full((1, D)),
            full((D, 1)), full((1, 1)),
            full((2 * D, D)), full((1, D)),
            full((D, 1)), full((1, 1))],
        out_specs=pl.BlockSpec((blk, 2), lambda i: (i, 0)),
        out_shape=jax.ShapeDtypeStruct((B, 2), _f32),
    )(*gathered,
      p["p_W1"].T, p["p_b1"][None, :],
      p["p_W2"].T, p["p_b2"][None, :],
      p["p_W3"].T, p["p_b3"][None, :],
      p["g_W1"].T, p["g_b1"][None, :],
      p["g_W2"].T, p["g_b2"][None, :])


def _prep_edges(ei, e_real, ep, pad_dst):
    src = jnp.pad(ei[0], (0, ep - e_real))
    dst = jnp.pad(ei[1], (0, ep - e_real), constant_values=pad_dst)
    return src.reshape(ep // 128, 128), dst.reshape(ep // 128, 128)


def _gcn_stack_run(xa, xb, src2, dst2, npad, ep, stacks, w2=None,
                   final_agg=None):
    n = len(stacks)
    cnt1 = None
    for l, (wsT, wnT, bias) in enumerate(stacks):
        seg = _segsum_kernel(npad, ep, l == 0, w2 is not None)
        args = (xa, xb, src2, dst2) + ((w2,) if w2 is not None else ())
        if l == 0:
            ga, gb, cnt = seg(*args)
            cnt1 = cnt.reshape(npad, 1)
        else:
            ga, gb = seg(*args)
        agg_w = final_agg if l == n - 1 else None
        xa, xb = _tc_combine(xa, xb, ga, gb, cnt1, wsT, wnT, bias, npad,
                             agg_w=agg_w)
    return xa, xb


def kernel(params, social_edge_weights, user_ids, item_ids,
           initiator_edge_index, participant_edge_index, social_edge_index):
    p = params

    all_emb = jnp.concatenate([p["user_table"], p["item_table"]], axis=0)
    all_pad = jnp.pad(all_emb, ((0, NPAD - N_ALL), (0, 0)))
    xa0, xb0 = all_pad[:, :32], all_pad[:, 32:]

    ut_pad = jnp.pad(p["user_table"], ((0, UPAD - U), (0, 0)))
    sa0, sb0 = ut_pad[:, :32], ut_pad[:, 32:]

    isrc2, idst2 = _prep_edges(initiator_edge_index, 800000, EPV, NPAD - 1)
    psrc2, pdst2 = _prep_edges(participant_edge_index, 800000, EPV, NPAD - 1)
    ssrc2, sdst2 = _prep_edges(social_edge_index, 400000, EPS, UPAD - 1)
    sw2 = jnp.pad(social_edge_weights, (0, EPS - 400000)).reshape(EPS // 128, 128)

    def stack(prefix, n):
        return [(p[prefix + "_Ws"][l].T, p[prefix + "_Wn"][l].T,
                 (p[prefix + "_bs"][l] + p[prefix + "_bn"][l]
                  + p[prefix + "_bias"][l])[None, :]) for l in range(n)]

    ia, ib = _gcn_stack_run(xa0, xb0, isrc2, idst2, NPAD, EPV,
                            stack("init", N_LAYERS))
    pa, pb = _gcn_stack_run(xa0, xb0, psrc2, pdst2, NPAD, EPV,
                            stack("part", N_LAYERS))
    sa, sb = _gcn_stack_run(sa0, sb0, ssrc2, sdst2, UPAD, EPS,
                            stack("soc", N_SOC_LAYERS), w2=sw2,
                            final_agg=(p["soc_agg_W"].T,
                                       p["soc_agg_b"][None, :]))

    uia, uib, upa, upb = _tc_cross(ia[:UPAD], ib[:UPAD], pa[:UPAD], pb[:UPAD], p)

    ui2 = user_ids.astype(_i32).reshape(B // 128, 128)
    it2 = item_ids.astype(_i32).reshape(B // 128, 128)
    gathered = _batch_gather_kernel()(ui2, it2, uia, uib, upa, upb,
                                      sa, sb, ia, ib, pa, pb)
    return _tc_head(gathered, p)


# speedup vs baseline: 8.9891x; 1.7974x over previous
import functools

import jax
import jax.numpy as jnp
from jax import lax
from jax.experimental import pallas as pl
from jax.experimental.pallas import tpu as pltpu
from jax.experimental.pallas import tpu_sc as plsc

U = 25000
I = 25000
D = 64
B = 16384
N_LAYERS = 3
N_SOC_LAYERS = 2
ALPHA = 0.6
BETA = 0.4

N_ALL = U + I
NPAD = 50176
UPAD = 25088
EPV = 802816
EPS = 409600

_f32 = jnp.float32
_i32 = jnp.int32


@functools.cache
def _segsum_kernel(npad, ep, with_counts, with_weights):
    nsub = 16
    stripe = npad // nsub
    rows_per_sub = ep // nsub // 128
    kj = 4
    nblk = rows_per_sub // kj
    rbuf = kj * 128
    nfull, rem = divmod(stripe, rbuf)
    wplan = [(t * rbuf, rbuf) for t in range(nfull)]
    if rem:
        wplan.append((nfull * rbuf, rem))

    mesh = plsc.VectorSubcoreMesh(core_axis_name="c", subcore_axis_name="s")

    out_type = [jax.ShapeDtypeStruct((npad, 32), _f32),
                jax.ShapeDtypeStruct((npad, 32), _f32)]
    if with_counts:
        out_type.append(jax.ShapeDtypeStruct((npad,), _f32))

    scratch = [
        pltpu.VMEM((kj, 128), _i32),
        pltpu.VMEM((kj, 128), _i32),
        pltpu.VMEM((rbuf, 32), _f32),
        pltpu.VMEM_SHARED((npad, 32), _f32),
        pltpu.SemaphoreType.DMA,
    ]
    if with_counts:
        scratch += [
            pltpu.VMEM((kj, 128), _f32),
            pltpu.VMEM_SHARED((npad,), _f32),
            pltpu.VMEM((stripe,), _f32),
        ]
    if with_weights:
        scratch.append(pltpu.VMEM((kj, 128), _f32))

    def body(*refs):
        it = iter(refs)
        xa = next(it); xb = next(it); src2 = next(it); dst2 = next(it)
        w2 = next(it) if with_weights else None
        ga = next(it); gb = next(it)
        cnt = next(it) if with_counts else None
        src_v = next(it); dst_v = next(it); rows_v = next(it)
        acc = next(it); sem = next(it)
        if with_counts:
            ones_v = next(it); cacc = next(it); ctmp = next(it)
        if with_weights:
            wts = next(it)

        cid = lax.axis_index("c")
        sid = lax.axis_index("s")

        zeros16 = jnp.zeros((16,), _f32)

        def zrow(r, carry):
            rows_v[r, pl.ds(0, 16)] = zeros16
            rows_v[r, pl.ds(16, 16)] = zeros16
            return carry
        lax.fori_loop(0, rbuf, zrow, 0)
        for off, ln in wplan:
            pltpu.sync_copy(rows_v.at[pl.ds(0, ln)],
                            acc.at[pl.ds(sid * stripe + off, ln)])

        if with_counts:
            def czrow(r, carry):
                ctmp[pl.ds(r * 16, 16)] = zeros16
                return carry
            lax.fori_loop(0, stripe // 16, czrow, 0)
            pltpu.sync_copy(ctmp, cacc.at[pl.ds(sid * stripe, stripe)])
            ones16 = jnp.ones((16,), _f32)
            for r in range(kj):
                for q in range(8):
                    ones_v[r, pl.ds(q * 16, 16)] = ones16

        plsc.subcore_barrier()

        def blk_body(b, carry):
            rbase = sid * rows_per_sub + b * kj
            pltpu.sync_copy(src2.at[pl.ds(rbase, kj)], src_v)
            pltpu.sync_copy(dst2.at[pl.ds(rbase, kj)], dst_v)
            if with_weights:
                pltpu.sync_copy(w2.at[pl.ds(rbase, kj)], wts)

            @pl.when(cid == 0)
            def _():
                for j in range(kj):
                    pltpu.async_copy(xa.at[src_v.at[j]],
                                     rows_v.at[pl.ds(j * 128, 128)], sem)

            @pl.when(cid == 1)
            def _():
                for j in range(kj):
                    pltpu.async_copy(xb.at[src_v.at[j]],
                                     rows_v.at[pl.ds(j * 128, 128)], sem)

            pltpu.make_async_copy(xa.at[pl.ds(0, kj * 128)], rows_v, sem).wait()

            if with_weights:
                def scale(g, carry):
                    w16 = wts[g // 8, pl.ds((g % 8) * 16, 16)]
                    base = g * 16
                    for t in range(16):
                        e = base + t
                        wv = jnp.full((16,), w16[t], _f32)
                        rows_v[e, pl.ds(0, 16)] = rows_v[e, pl.ds(0, 16)] * wv
                        rows_v[e, pl.ds(16, 16)] = rows_v[e, pl.ds(16, 16)] * wv
                    return carry
                lax.fori_loop(0, kj * 8, scale, 0)

            for j in range(kj):
                pltpu.sync_copy(rows_v.at[pl.ds(j * 128, 128)],
                                acc.at[dst_v.at[j]], add=True)
            if with_counts:
                for j in range(kj):
                    pltpu.sync_copy(ones_v.at[j], cacc.at[dst_v.at[j]], add=True)
            return carry

        lax.fori_loop(0, nblk, blk_body, 0)

        plsc.subcore_barrier()

        for off, ln in wplan:
            o = sid * stripe + off
            pltpu.sync_copy(acc.at[pl.ds(o, ln)], rows_v.at[pl.ds(0, ln)])

            @pl.when(cid == 0)
            def _():
                pltpu.sync_copy(rows_v.at[pl.ds(0, ln)], ga.at[pl.ds(o, ln)])

            @pl.when(cid == 1)
            def _():
                pltpu.sync_copy(rows_v.at[pl.ds(0, ln)], gb.at[pl.ds(o, ln)])

        if with_counts:
            pltpu.sync_copy(cacc.at[pl.ds(sid * stripe, stripe)], ctmp)

            @pl.when(cid == 0)
            def _():
                pltpu.sync_copy(ctmp, cnt.at[pl.ds(sid * stripe, stripe)])

    return pl.kernel(body, out_type=out_type, mesh=mesh,
                     scratch_types=scratch,
                     compiler_params=pltpu.CompilerParams(
                         use_tc_tiling_on_sc=False,
                         needs_layout_passes=False))


@functools.cache
def _batch_gather_kernel():
    mesh = plsc.VectorSubcoreMesh(core_axis_name="c", subcore_axis_name="s")
    out_type = [jax.ShapeDtypeStruct((B, 32), _f32) for _ in range(10)]
    scratch = [
        pltpu.VMEM((8, 128), _i32),
        pltpu.VMEM((8, 128), _i32),
        pltpu.VMEM((1024, 32), _f32),
        pltpu.SemaphoreType.DMA,
    ]

    def body(ui2, it2, uia, uib, upa, upb, sia, sib, ia, ib, pa, pb,
             fua, fub, pua, pub, sua, sub2, iia, iib, pia, pib,
             uid_v, iid_v, rows_v, sem):
        cid = lax.axis_index("c")
        sid = lax.axis_index("s")
        rbase = sid * 8
        pltpu.sync_copy(ui2.at[pl.ds(rbase, 8)], uid_v)
        pltpu.sync_copy(it2.at[pl.ds(rbase, 8)], iid_v)
        for r in range(8):
            for q in range(8):
                iid_v[r, pl.ds(q * 16, 16)] = iid_v[r, pl.ds(q * 16, 16)] + U

        def one(ta, tb, oa, ob, idx_v):
            @pl.when(cid == 0)
            def _():
                for j in range(8):
                    pltpu.async_copy(ta.at[idx_v.at[j]],
                                     rows_v.at[pl.ds(j * 128, 128)], sem)

            @pl.when(cid == 1)
            def _():
                for j in range(8):
                    pltpu.async_copy(tb.at[idx_v.at[j]],
                                     rows_v.at[pl.ds(j * 128, 128)], sem)

            pltpu.make_async_copy(ta.at[pl.ds(0, 1024)], rows_v, sem).wait()

            @pl.when(cid == 0)
            def _():
                pltpu.sync_copy(rows_v, oa.at[pl.ds(sid * 1024, 1024)])

            @pl.when(cid == 1)
            def _():
                pltpu.sync_copy(rows_v, ob.at[pl.ds(sid * 1024, 1024)])

        one(uia, uib, fua, fub, uid_v)
        one(upa, upb, pua, pub, uid_v)
        one(sia, sib, sua, sub2, uid_v)
        one(ia, ib, iia, iib, iid_v)
        one(pa, pb, pia, pib, iid_v)

    return pl.kernel(body, out_type=out_type, mesh=mesh,
                     scratch_types=scratch,
                     compiler_params=pltpu.CompilerParams(
                         use_tc_tiling_on_sc=False))


def _leaky(x):
    return jnp.where(x >= 0, x, 0.2 * x)


def _combine_body(xa, xb, ga, gb, cnt, wsT, wnT, bias, oa, ob):
    x = jnp.concatenate([xa[...], xb[...]], axis=1)
    gs = jnp.concatenate([ga[...], gb[...]], axis=1) + x
    agg = gs / (cnt[...] + 1.0)
    y = _leaky(x @ wsT[...] + agg @ wnT[...] + bias[...])
    oa[...] = y[:, :32]
    ob[...] = y[:, 32:]


def _combine_agg_body(xa, xb, ga, gb, cnt, wsT, wnT, bias, waggT, bagg, oa, ob):
    x = jnp.concatenate([xa[...], xb[...]], axis=1)
    gs = jnp.concatenate([ga[...], gb[...]], axis=1) + x
    agg = gs / (cnt[...] + 1.0)
    y = _leaky(x @ wsT[...] + agg @ wnT[...] + bias[...])
    s = y @ waggT[...] + bagg[...]
    oa[...] = s[:, :32]
    ob[...] = s[:, 32:]


def _tc_combine(xa, xb, ga, gb, cnt1, wsT, wnT, bias, npad, agg_w=None):
    blk = 3136
    grid = (npad // blk,)
    half = lambda: pl.BlockSpec((blk, 32), lambda i: (i, 0))
    full = lambda shape: pl.BlockSpec(shape, lambda i: (0, 0))
    in_specs = [half(), half(), half(), half(),
                pl.BlockSpec((blk, 1), lambda i: (i, 0)),
                full((D, D)), full((D, D)), full((1, D))]
    args = [xa, xb, ga, gb, cnt1, wsT, wnT, bias]
    body = _combine_body
    if agg_w is not None:
        in_specs += [full((D, D)), full((1, D))]
        args += [agg_w[0], agg_w[1]]
        body = _combine_agg_body
    return pl.pallas_call(
        body,
        grid=grid,
        in_specs=in_specs,
        out_specs=[pl.BlockSpec((blk, 32), lambda i: (i, 0))] * 2,
        out_shape=[jax.ShapeDtypeStruct((npad, 32), _f32)] * 2,
    )(*args)


def _cross_body(ia, ib, pa, pb, waiT, bai, wapT, bap, wcT, bc, wxT, bx,
                oia, oib, opa, opb):
    iu = jnp.concatenate([ia[...], ib[...]], axis=1)
    pu = jnp.concatenate([pa[...], pb[...]], axis=1)
    att_i = iu @ waiT[...] + bai[...]
    att_p = pu @ wapT[...] + bap[...]
    z = jnp.concatenate([att_i, att_p], axis=1) @ wcT[...] + bc[...]
    aw = jax.nn.sigmoid(z)
    px = pu @ wxT[...] + bx[...]
    ix = iu @ wxT[...] + bx[...]
    ui = iu + aw * px
    up = pu + (1.0 - aw) * ix
    oia[...] = ui[:, :32]
    oib[...] = ui[:, 32:]
    opa[...] = up[:, :32]
    opb[...] = up[:, 32:]


def _tc_cross(ia, ib, pa, pb, p):
    blk = 3136
    grid = (UPAD // blk,)
    half = lambda: pl.BlockSpec((blk, 32), lambda i: (i, 0))
    full = lambda shape: pl.BlockSpec(shape, lambda i: (0, 0))
    return pl.pallas_call(
        _cross_body,
        grid=grid,
        in_specs=[half(), half(), half(), half(),
                  full((D, D)), full((1, D)),
                  full((D, D)), full((1, D)),
                  full((2 * D, 1)), full((1, 1)),
                  full((D, D)), full((1, D))],
        out_specs=[pl.BlockSpec((blk, 32), lambda i: (i, 0))] * 4,
        out_shape=[jax.ShapeDtypeStruct((UPAD, 32), _f32)] * 4,
    )(ia, ib, pa, pb,
      p["cv_Wai"].T, p["cv_bai"][None, :],
      p["cv_Wap"].T, p["cv_bap"][None, :],
      p["cv_Wc"].T, p["cv_bc"][None, :],
      p["cv_Wx"].T, p["cv_bx"][None, :])


def _head_body(fua, fub, pua, pub, sua, sub2, iia, iib, pia, pib,
               w1, b1, w2, b2, w3, b3, gw1, gb1, gw2, gb2, out):
    fu = jnp.concatenate([fua[...], fub[...]], axis=1)
    pu = jnp.concatenate([pua[...], pub[...]], axis=1)
    su = jnp.concatenate([sua[...], sub2[...]], axis=1)
    itf = ALPHA * jnp.concatenate([iia[...], iib[...]], axis=1) + \
        BETA * jnp.concatenate([pia[...], pib[...]], axis=1)
    feat = jnp.concatenate([fu, pu, su, itf], axis=1)
    h = jnp.maximum(feat @ w1[...] + b1[...], 0.0)
    h = jnp.maximum(h @ w2[...] + b2[...], 0.0)
    pred = jax.nn.sigmoid(h @ w3[...] + b3[...])
    g = jnp.concatenate([fu, itf], axis=1)
    gh = jnp.maximum(g @ gw1[...] + gb1[...], 0.0)
    group = jax.nn.sigmoid(gh @ gw2[...] + gb2[...])
    out[...] = jnp.concatenate([pred, group], axis=1)


def _tc_head(gathered, p):
    blk = 2048
    grid = (B // blk,)
    half = lambda: pl.BlockSpec((blk, 32), lambda i: (i, 0))
    full = lambda shape: pl.BlockSpec(shape, lambda i: (0, 0))
    return pl.pallas_call(
        _head_body,
        grid=grid,
        in_specs=[half() for _ in range(10)] + [
            full((4 * D, 2 * D)), full((1, 2 * D)),
            full((2 * D, D)), full((1, D)),
            full((D, 1)), full((1, 1)),
            full((2 * D, D)), full((1, D)),
            full((D, 1)), full((1, 1))],
        out_specs=pl.BlockSpec((blk, 2), lambda i: (i, 0)),
        out_shape=jax.ShapeDtypeStruct((B, 2), _f32),
    )(*gathered,
      p["p_W1"].T, p["p_b1"][None, :],
      p["p_W2"].T, p["p_b2"][None, :],
      p["p_W3"].T, p["p_b3"][None, :],
      p["g_W1"].T, p["g_b1"][None, :],
      p["g_W2"].T, p["g_b2"][None, :])


def _prep_edges(ei, e_real, ep, pad_dst):
    src = jnp.pad(ei[0], (0, ep - e_real))
    dst = jnp.pad(ei[1], (0, ep - e_real), constant_values=pad_dst)
    return src.reshape(ep // 128, 128), dst.reshape(ep // 128, 128)


def _gcn_stack_run(xa, xb, src2, dst2, npad, ep, stacks, w2=None,
                   final_agg=None):
    n = len(stacks)
    cnt1 = None
    for l, (wsT, wnT, bias) in enumerate(stacks):
        seg = _segsum_kernel(npad, ep, l == 0, w2 is not None)
        args = (xa, xb, src2, dst2) + ((w2,) if w2 is not None else ())
        if l == 0:
            ga, gb, cnt = seg(*args)
            cnt1 = cnt.reshape(npad, 1)
        else:
            ga, gb = seg(*args)
        agg_w = final_agg if l == n - 1 else None
        xa, xb = _tc_combine(xa, xb, ga, gb, cnt1, wsT, wnT, bias, npad,
                             agg_w=agg_w)
    return xa, xb


def kernel(params, social_edge_weights, user_ids, item_ids,
           initiator_edge_index, participant_edge_index, social_edge_index):
    p = params

    all_emb = jnp.concatenate([p["user_table"], p["item_table"]], axis=0)
    all_pad = jnp.pad(all_emb, ((0, NPAD - N_ALL), (0, 0)))
    xa0, xb0 = all_pad[:, :32], all_pad[:, 32:]

    ut_pad = jnp.pad(p["user_table"], ((0, UPAD - U), (0, 0)))
    sa0, sb0 = ut_pad[:, :32], ut_pad[:, 32:]

    isrc2, idst2 = _prep_edges(initiator_edge_index, 800000, EPV, NPAD - 1)
    psrc2, pdst2 = _prep_edges(participant_edge_index, 800000, EPV, NPAD - 1)
    ssrc2, sdst2 = _prep_edges(social_edge_index, 400000, EPS, UPAD - 1)
    sw2 = jnp.pad(social_edge_weights, (0, EPS - 400000)).reshape(EPS // 128, 128)

    def stack(prefix, n):
        return [(p[prefix + "_Ws"][l].T, p[prefix + "_Wn"][l].T,
                 (p[prefix + "_bs"][l] + p[prefix + "_bn"][l]
                  + p[prefix + "_bias"][l])[None, :]) for l in range(n)]

    ia, ib = _gcn_stack_run(xa0, xb0, isrc2, idst2, NPAD, EPV,
                            stack("init", N_LAYERS))
    pa, pb = _gcn_stack_run(xa0, xb0, psrc2, pdst2, NPAD, EPV,
                            stack("part", N_LAYERS))
    sa, sb = _gcn_stack_run(sa0, sb0, ssrc2, sdst2, UPAD, EPS,
                            stack("soc", N_SOC_LAYERS), w2=sw2,
                            final_agg=(p["soc_agg_W"].T,
                                       p["soc_agg_b"][None, :]))

    uia, uib, upa, upb = _tc_cross(ia[:UPAD], ib[:UPAD], pa[:UPAD], pb[:UPAD], p)

    ui2 = user_ids.astype(_i32).reshape(B // 128, 128)
    it2 = item_ids.astype(_i32).reshape(B // 128, 128)
    gathered = _batch_gather_kernel()(ui2, it2, uia, uib, upa, upb,
                                      sa, sb, ia, ib, pa, pb)
    return _tc_head(gathered, p)


# speedup vs baseline: 10.5536x; 1.1740x over previous
import functools

import jax
import jax.numpy as jnp
from jax import lax
from jax.experimental import pallas as pl
from jax.experimental.pallas import tpu as pltpu
from jax.experimental.pallas import tpu_sc as plsc

U = 25000
I = 25000
D = 64
B = 16384
N_LAYERS = 3
N_SOC_LAYERS = 2
ALPHA = 0.6
BETA = 0.4

N_ALL = U + I
NPAD = 50176
UPAD = 25088
EPV = 811008
EPS = 405504

_f32 = jnp.float32
_i32 = jnp.int32

_SC_PARAMS = pltpu.CompilerParams(use_tc_tiling_on_sc=False,
                                  needs_layout_passes=False)


@functools.cache
def _segsum_kernel(npad, ep, with_weights):
    nsub = 16
    stripe = npad // nsub
    rows_per_sub = ep // nsub // 128
    kj = 3
    rbuf = kj * 128
    nblk = rows_per_sub // kj
    npair = nblk // 2
    nfull, rem = divmod(stripe, rbuf)
    wplan = [(t * rbuf, rbuf) for t in range(nfull)]
    if rem:
        wplan.append((nfull * rbuf, rem))

    mesh = plsc.VectorSubcoreMesh(core_axis_name="c", subcore_axis_name="s")
    out_type = [jax.ShapeDtypeStruct((npad, 32), _f32)] * 2
    scratch = [
        pltpu.VMEM((kj, 2, 128), _i32), pltpu.VMEM((kj, 2, 128), _i32),
        pltpu.VMEM((rbuf, 32), _f32), pltpu.VMEM((rbuf, 32), _f32),
        pltpu.VMEM_SHARED((npad, 32), _f32),
        pltpu.SemaphoreType.DMA, pltpu.SemaphoreType.DMA,
        pltpu.SemaphoreType.DMA, pltpu.SemaphoreType.DMA,
    ]
    if with_weights:
        scratch += [pltpu.VMEM((kj, 128), _f32), pltpu.VMEM((kj, 128), _f32)]

    def body(*refs):
        it = iter(refs)
        xa = next(it); xb = next(it); ei2 = next(it)
        w2 = next(it) if with_weights else None
        ga = next(it); gb = next(it)
        idx = [next(it), next(it)]
        rows = [next(it), next(it)]
        acc = next(it)
        semg = [next(it), next(it)]
        sems = [next(it), next(it)]
        wts = [next(it), next(it)] if with_weights else None

        cid = lax.axis_index("c")
        sid = lax.axis_index("s")
        zeros16 = jnp.zeros((16,), _f32)

        def zrow(r, carry):
            rows[0][r, pl.ds(0, 16)] = zeros16
            rows[0][r, pl.ds(16, 16)] = zeros16
            return carry
        lax.fori_loop(0, rbuf, zrow, 0)
        for off, ln in wplan:
            pltpu.sync_copy(rows[0].at[pl.ds(0, ln)],
                            acc.at[pl.ds(sid * stripe + off, ln)])
        plsc.subcore_barrier()

        def load_idx(b, s):
            rbase = sid * rows_per_sub + b * kj
            pltpu.sync_copy(ei2.at[pl.ds(rbase, kj)], idx[s])
            if with_weights:
                pltpu.sync_copy(w2.at[pl.ds(rbase, kj)], wts[s])

        def fire_gathers(s):
            @pl.when(cid == 0)
            def _():
                for j in range(kj):
                    pltpu.async_copy(xa.at[idx[s].at[j, 0]],
                                     rows[s].at[pl.ds(j * 128, 128)], semg[s])

            @pl.when(cid == 1)
            def _():
                for j in range(kj):
                    pltpu.async_copy(xb.at[idx[s].at[j, 0]],
                                     rows[s].at[pl.ds(j * 128, 128)], semg[s])

        def wait_gathers(s):
            pltpu.make_async_copy(xa.at[pl.ds(0, rbuf)], rows[s],
                                  semg[s]).wait()

        def scale(s):
            def go(g, carry):
                w16 = wts[s][g // 8, pl.ds((g % 8) * 16, 16)]
                base = g * 16
                for t in range(16):
                    e = base + t
                    wv = jnp.full((16,), w16[t], _f32)
                    rows[s][e, pl.ds(0, 16)] = rows[s][e, pl.ds(0, 16)] * wv
                    rows[s][e, pl.ds(16, 16)] = rows[s][e, pl.ds(16, 16)] * wv
                return carry
            lax.fori_loop(0, kj * 8, go, 0)

        def fire_scatters(s):
            if with_weights:
                scale(s)
            for j in range(kj):
                pltpu.async_copy(rows[s].at[pl.ds(j * 128, 128)],
                                 acc.at[idx[s].at[j, 1]], sems[s], add=True)

        def wait_scatters(s):
            pltpu.make_async_copy(rows[s], acc.at[pl.ds(0, rbuf)],
                                  sems[s]).wait()

        load_idx(0, 0)
        fire_gathers(0)

        def pair(i, carry):
            b0 = 2 * i

            @pl.when(i > 0)
            def _():
                wait_scatters(1)
            load_idx(b0 + 1, 1)
            fire_gathers(1)

            wait_gathers(0)
            fire_scatters(0)

            @pl.when(i < npair - 1)
            def _():
                wait_scatters(0)
                load_idx(b0 + 2, 0)
                fire_gathers(0)

            wait_gathers(1)
            fire_scatters(1)
            return carry

        lax.fori_loop(0, npair, pair, 0)
        wait_scatters(0)
        wait_scatters(1)
        plsc.subcore_barrier()

        for off, ln in wplan:
            o = sid * stripe + off
            pltpu.sync_copy(acc.at[pl.ds(o, ln)], rows[0].at[pl.ds(0, ln)])

            @pl.when(cid == 0)
            def _():
                pltpu.sync_copy(rows[0].at[pl.ds(0, ln)], ga.at[pl.ds(o, ln)])

            @pl.when(cid == 1)
            def _():
                pltpu.sync_copy(rows[0].at[pl.ds(0, ln)], gb.at[pl.ds(o, ln)])

    return pl.kernel(body, out_type=out_type, mesh=mesh,
                     scratch_types=scratch, compiler_params=_SC_PARAMS)


_CNT_CFG = (
    ("i", NPAD, EPV, 6),
    ("p", NPAD, EPV, 6),
    ("s", UPAD, EPS, 9),
)


@functools.cache
def _counts_kernel():
    mesh = plsc.VectorSubcoreMesh(core_axis_name="c", subcore_axis_name="s")
    out_type = [jax.ShapeDtypeStruct((2 * npad,), _f32)
                for _, npad, _, _ in _CNT_CFG]
    kjmax = max(k for _, _, _, k in _CNT_CFG)
    ctmax = max(npad // 16 for _, npad, _, _ in _CNT_CFG)
    scratch = [
        pltpu.VMEM((kjmax, 128), _i32),
        pltpu.VMEM((kjmax * 128,), _f32),
        pltpu.VMEM((ctmax,), _f32),
        pltpu.SemaphoreType.DMA,
    ] + [pltpu.VMEM_SHARED((npad,), _f32) for _, npad, _, _ in _CNT_CFG]

    def body(idst2, pdst2, sdst2, ci, cp, cs, idx_v, ones_v, ctmp, sem,
             acc_i, acc_p, acc_s):
        cid = lax.axis_index("c")
        sid = lax.axis_index("s")
        zeros16 = jnp.zeros((16,), _f32)
        ones16 = jnp.ones((16,), _f32)

        def fill_z(r, carry):
            ctmp[pl.ds(r * 16, 16)] = zeros16
            return carry
        lax.fori_loop(0, ctmax // 16, fill_z, 0)

        def fill_o(r, carry):
            ones_v[pl.ds(r * 16, 16)] = ones16
            return carry
        lax.fori_loop(0, kjmax * 128 // 16, fill_o, 0)

        for (dst2, acc, out, (_, npad, ep, kjc)) in (
                (idst2, acc_i, ci, _CNT_CFG[0]),
                (pdst2, acc_p, cp, _CNT_CFG[1]),
                (sdst2, acc_s, cs, _CNT_CFG[2])):
            stripe = npad // 16
            pltpu.sync_copy(ctmp.at[pl.ds(0, stripe)],
                            acc.at[pl.ds(sid * stripe, stripe)])
            plsc.subcore_barrier()
            rows_tot = ep // 128
            rps = rows_tot // 32
            nb = rps // kjc
            rbase0 = cid * (rows_tot // 2) + sid * rps

            def blk(b, carry):
                pltpu.sync_copy(dst2.at[pl.ds(rbase0 + b * kjc, kjc)],
                                idx_v.at[pl.ds(0, kjc)])
                for j in range(kjc):
                    pltpu.async_copy(ones_v.at[pl.ds(j * 128, 128)],
                                     acc.at[idx_v.at[j]], sem, add=True)
                pltpu.make_async_copy(ones_v.at[pl.ds(0, kjc * 128)],
                                      acc.at[pl.ds(0, kjc * 128)], sem).wait()
                return carry
            lax.fori_loop(0, nb, blk, 0)
            plsc.subcore_barrier()
            pltpu.sync_copy(acc.at[pl.ds(sid * stripe, stripe)],
                            ctmp.at[pl.ds(0, stripe)])

            def wb(coff):
                pltpu.sync_copy(ctmp.at[pl.ds(0, stripe)],
                                out.at[pl.ds(coff + sid * stripe, stripe)])

            @pl.when(cid == 0)
            def _():
                wb(0)

            @pl.when(cid == 1)
            def _():
                wb(npad)
            def refill(r, carry):
                ctmp[pl.ds(r * 16, 16)] = zeros16
                return carry
            lax.fori_loop(0, stripe // 16, refill, 0)

    return pl.kernel(body, out_type=out_type, mesh=mesh,
                     scratch_types=scratch, compiler_params=_SC_PARAMS)


@functools.cache
def _batch_gather_kernel():
    mesh = plsc.VectorSubcoreMesh(core_axis_name="c", subcore_axis_name="s")
    out_type = [jax.ShapeDtypeStruct((B, 32), _f32) for _ in range(10)]
    scratch = [
        pltpu.VMEM((8, 128), _i32),
        pltpu.VMEM((8, 128), _i32),
        pltpu.VMEM((1024, 32), _f32),
        pltpu.SemaphoreType.DMA,
    ]

    def body(ui2, it2, uia, uib, upa, upb, sia, sib, ia, ib, pa, pb,
             fua, fub, pua, pub, sua, sub2, iia, iib, pia, pib,
             uid_v, iid_v, rows_v, sem):
        cid = lax.axis_index("c")
        sid = lax.axis_index("s")
        rbase = sid * 8
        pltpu.sync_copy(ui2.at[pl.ds(rbase, 8)], uid_v)
        pltpu.sync_copy(it2.at[pl.ds(rbase, 8)], iid_v)
        for r in range(8):
            for q in range(8):
                iid_v[r, pl.ds(q * 16, 16)] = iid_v[r, pl.ds(q * 16, 16)] + U

        def one(ta, tb, oa, ob, idx_v):
            @pl.when(cid == 0)
            def _():
                for j in range(8):
                    pltpu.async_copy(ta.at[idx_v.at[j]],
                                     rows_v.at[pl.ds(j * 128, 128)], sem)

            @pl.when(cid == 1)
            def _():
                for j in range(8):
                    pltpu.async_copy(tb.at[idx_v.at[j]],
                                     rows_v.at[pl.ds(j * 128, 128)], sem)

            pltpu.make_async_copy(ta.at[pl.ds(0, 1024)], rows_v, sem).wait()

            @pl.when(cid == 0)
            def _():
                pltpu.sync_copy(rows_v, oa.at[pl.ds(sid * 1024, 1024)])

            @pl.when(cid == 1)
            def _():
                pltpu.sync_copy(rows_v, ob.at[pl.ds(sid * 1024, 1024)])

        one(uia, uib, fua, fub, uid_v)
        one(upa, upb, pua, pub, uid_v)
        one(sia, sib, sua, sub2, uid_v)
        one(ia, ib, iia, iib, iid_v)
        one(pa, pb, pia, pib, iid_v)

    return pl.kernel(body, out_type=out_type, mesh=mesh,
                     scratch_types=scratch, compiler_params=_SC_PARAMS)


def _leaky(x):
    return jnp.where(x >= 0, x, 0.2 * x)


def _combine_body(xa, xb, ga, gb, c0, c1, wsT, wnT, bias, oa, ob):
    x = jnp.concatenate([xa[...], xb[...]], axis=1)
    gs = jnp.concatenate([ga[...], gb[...]], axis=1) + x
    agg = gs / (c0[...] + c1[...] + 1.0)
    y = _leaky(x @ wsT[...] + agg @ wnT[...] + bias[...])
    oa[...] = y[:, :32]
    ob[...] = y[:, 32:]


def _combine_agg_body(xa, xb, ga, gb, c0, c1, wsT, wnT, bias, waggT, bagg,
                      oa, ob):
    x = jnp.concatenate([xa[...], xb[...]], axis=1)
    gs = jnp.concatenate([ga[...], gb[...]], axis=1) + x
    agg = gs / (c0[...] + c1[...] + 1.0)
    y = _leaky(x @ wsT[...] + agg @ wnT[...] + bias[...])
    s = y @ waggT[...] + bagg[...]
    oa[...] = s[:, :32]
    ob[...] = s[:, 32:]


def _tc_combine(xa, xb, ga, gb, c0, c1, wsT, wnT, bias, npad, agg_w=None):
    blk = 3136
    grid = (npad // blk,)
    half = lambda: pl.BlockSpec((blk, 32), lambda i: (i, 0))
    cspec = lambda: pl.BlockSpec((blk, 1), lambda i: (i, 0))
    full = lambda shape: pl.BlockSpec(shape, lambda i: (0, 0))
    in_specs = [half(), half(), half(), half(), cspec(), cspec(),
                full((D, D)), full((D, D)), full((1, D))]
    args = [xa, xb, ga, gb, c0, c1, wsT, wnT, bias]
    body = _combine_body
    if agg_w is not None:
        in_specs += [full((D, D)), full((1, D))]
        args += [agg_w[0], agg_w[1]]
        body = _combine_agg_body
    return pl.pallas_call(
        body,
        grid=grid,
        in_specs=in_specs,
        out_specs=[pl.BlockSpec((blk, 32), lambda i: (i, 0))] * 2,
        out_shape=[jax.ShapeDtypeStruct((npad, 32), _f32)] * 2,
    )(*args)


def _cross_body(ia, ib, pa, pb, waiT, bai, wapT, bap, wcT, bc, wxT, bx,
                oia, oib, opa, opb):
    iu = jnp.concatenate([ia[...], ib[...]], axis=1)
    pu = jnp.concatenate([pa[...], pb[...]], axis=1)
    att_i = iu @ waiT[...] + bai[...]
    att_p = pu @ wapT[...] + bap[...]
    z = jnp.concatenate([att_i, att_p], axis=1) @ wcT[...] + bc[...]
    aw = jax.nn.sigmoid(z)
    px = pu @ wxT[...] + bx[...]
    ix = iu @ wxT[...] + bx[...]
    ui = iu + aw * px
    up = pu + (1.0 - aw) * ix
    oia[...] = ui[:, :32]
    oib[...] = ui[:, 32:]
    opa[...] = up[:, :32]
    opb[...] = up[:, 32:]


def _tc_cross(ia, ib, pa, pb, p):
    blk = 3136
    grid = (UPAD // blk,)
    half = lambda: pl.BlockSpec((blk, 32), lambda i: (i, 0))
    full = lambda shape: pl.BlockSpec(shape, lambda i: (0, 0))
    return pl.pallas_call(
        _cross_body,
        grid=grid,
        in_specs=[half(), half(), half(), half(),
                  full((D, D)), full((1, D)),
                  full((D, D)), full((1, D)),
                  full((2 * D, 1)), full((1, 1)),
                  full((D, D)), full((1, D))],
        out_specs=[pl.BlockSpec((blk, 32), lambda i: (i, 0))] * 4,
        out_shape=[jax.ShapeDtypeStruct((UPAD, 32), _f32)] * 4,
    )(ia, ib, pa, pb,
      p["cv_Wai"].T, p["cv_bai"][None, :],
      p["cv_Wap"].T, p["cv_bap"][None, :],
      p["cv_Wc"].T, p["cv_bc"][None, :],
      p["cv_Wx"].T, p["cv_bx"][None, :])


def _head_body(fua, fub, pua, pub, sua, sub2, iia, iib, pia, pib,
               w1, b1, w2, b2, w3, b3, gw1, gb1, gw2, gb2, out):
    fu = jnp.concatenate([fua[...], fub[...]], axis=1)
    pu = jnp.concatenate([pua[...], pub[...]], axis=1)
    su = jnp.concatenate([sua[...], sub2[...]], axis=1)
    itf = ALPHA * jnp.concatenate([iia[...], iib[...]], axis=1) + \
        BETA * jnp.concatenate([pia[...], pib[...]], axis=1)
    feat = jnp.concatenate([fu, pu, su, itf], axis=1)
    h = jnp.maximum(feat @ w1[...] + b1[...], 0.0)
    h = jnp.maximum(h @ w2[...] + b2[...], 0.0)
    pred = jax.nn.sigmoid(h @ w3[...] + b3[...])
    g = jnp.concatenate([fu, itf], axis=1)
    gh = jnp.maximum(g @ gw1[...] + gb1[...], 0.0)
    group = jax.nn.sigmoid(gh @ gw2[...] + gb2[...])
    out[...] = jnp.concatenate([pred, group], axis=1)


def _tc_head(gathered, p):
    blk = 2048
    grid = (B // blk,)
    half = lambda: pl.BlockSpec((blk, 32), lambda i: (i, 0))
    full = lambda shape: pl.BlockSpec(shape, lambda i: (0, 0))
    return pl.pallas_call(
        _head_body,
        grid=grid,
        in_specs=[half() for _ in range(10)] + [
            full((4 * D, 2 * D)), full((1, 2 * D)),
            full((2 * D, D)), full((1, D)),
            full((D, 1)), full((1, 1)),
            full((2 * D, D)), full((1, D)),
            full((D, 1)), full((1, 1))],
        out_specs=pl.BlockSpec((blk, 2), lambda i: (i, 0)),
        out_shape=jax.ShapeDtypeStruct((B, 2), _f32),
    )(*gathered,
      p["p_W1"].T, p["p_b1"][None, :],
      p["p_W2"].T, p["p_b2"][None, :],
      p["p_W3"].T, p["p_b3"][None, :],
      p["g_W1"].T, p["g_b1"][None, :],
      p["g_W2"].T, p["g_b2"][None, :])


def _prep_edges(ei, e_real, ep, pad_dst):
    src = jnp.pad(ei[0], (0, ep - e_real))
    dst = jnp.pad(ei[1], (0, ep - e_real), constant_values=pad_dst)
    src2 = src.reshape(ep // 128, 128)
    dst2 = dst.reshape(ep // 128, 128)
    ei2 = jnp.stack([src2, dst2], axis=1)
    return ei2, dst2


def _gcn_stack_run(xa, xb, ei2, c0, c1, npad, ep, stacks, w2=None,
                   final_agg=None):
    n = len(stacks)
    for l, (wsT, wnT, bias) in enumerate(stacks):
        seg = _segsum_kernel(npad, ep, w2 is not None)
        args = (xa, xb, ei2) + ((w2,) if w2 is not None else ())
        ga, gb = seg(*args)
        agg_w = final_agg if l == n - 1 else None
        xa, xb = _tc_combine(xa, xb, ga, gb, c0, c1, wsT, wnT, bias, npad,
                             agg_w=agg_w)
    return xa, xb


def kernel(params, social_edge_weights, user_ids, item_ids,
           initiator_edge_index, participant_edge_index, social_edge_index):
    p = params

    all_emb = jnp.concatenate([p["user_table"], p["item_table"]], axis=0)
    all_pad = jnp.pad(all_emb, ((0, NPAD - N_ALL), (0, 0)))
    xa0, xb0 = all_pad[:, :32], all_pad[:, 32:]

    ut_pad = jnp.pad(p["user_table"], ((0, UPAD - U), (0, 0)))
    sa0, sb0 = ut_pad[:, :32], ut_pad[:, 32:]

    iei2, idst2 = _prep_edges(initiator_edge_index, 800000, EPV, NPAD - 1)
    pei2, pdst2 = _prep_edges(participant_edge_index, 800000, EPV, NPAD - 1)
    sei2, sdst2 = _prep_edges(social_edge_index, 400000, EPS, UPAD - 1)
    sw2 = jnp.pad(social_edge_weights, (0, EPS - 400000)).reshape(EPS // 128, 128)

    ci, cp, cs = _counts_kernel()(idst2, pdst2, sdst2)
    ci0, ci1 = ci[:NPAD, None], ci[NPAD:, None]
    cp0, cp1 = cp[:NPAD, None], cp[NPAD:, None]
    cs0, cs1 = cs[:UPAD, None], cs[UPAD:, None]

    def stack(prefix, n):
        return [(p[prefix + "_Ws"][l].T, p[prefix + "_Wn"][l].T,
                 (p[prefix + "_bs"][l] + p[prefix + "_bn"][l]
                  + p[prefix + "_bias"][l])[None, :]) for l in range(n)]

    ia, ib = _gcn_stack_run(xa0, xb0, iei2, ci0, ci1, NPAD, EPV,
                            stack("init", N_LAYERS))
    pa, pb = _gcn_stack_run(xa0, xb0, pei2, cp0, cp1, NPAD, EPV,
                            stack("part", N_LAYERS))
    sa, sb = _gcn_stack_run(sa0, sb0, sei2, cs0, cs1, UPAD, EPS,
                            stack("soc", N_SOC_LAYERS), w2=sw2,
                            final_agg=(p["soc_agg_W"].T,
                                       p["soc_agg_b"][None, :]))

    uia, uib, upa, upb = _tc_cross(ia[:UPAD], ib[:UPAD], pa[:UPAD], pb[:UPAD], p)

    ui2 = user_ids.astype(_i32).reshape(B // 128, 128)
    it2 = item_ids.astype(_i32).reshape(B // 128, 128)
    gathered = _batch_gather_kernel()(ui2, it2, uia, uib, upa, upb,
                                      sa, sb, ia, ib, pa, pb)
    return _tc_head(gathered, p)
